# baseline (speedup 1.0000x reference)
import functools

import jax
import jax.numpy as jnp
from jax import lax
from jax.experimental import pallas as pl
from jax.experimental.pallas import tpu as pltpu
from jax.experimental.pallas import tpu_sc as plsc

N = 10000
E = 320000
D = 128

NC = 2
NS = 16
NW = NC * NS
NPAD = 10240
NPT = NPAD // NS
KE = 80
EPW = E // NW
CPW = EPW // KE
CPWP = 128
HB = 64

_mesh = plsc.VectorSubcoreMesh(core_axis_name="c", subcore_axis_name="s")


@functools.partial(
    pl.kernel,
    out_type=jax.ShapeDtypeStruct((2 * NPAD,), jnp.float32),
    mesh=_mesh,
    scratch_types=[
        pltpu.VMEM((CPWP, KE), jnp.int32),
        pltpu.VMEM((KE,), jnp.float32),
        pltpu.VMEM_SHARED((NPAD,), jnp.float32),
        pltpu.SemaphoreType.DMA,
        pltpu.SemaphoreType.DMA,
    ],
)
def _deg(dstr_hbm, ones_hbm, zeros_hbm, ones1_hbm, deg_out,
         idx_d, ones_v, deg_sh, sem0, sem1):
    c = lax.axis_index("c")
    s = lax.axis_index("s")
    wid = s * NC + c

    @pl.when(c == 0)
    def _():
        pltpu.sync_copy(ones_hbm, deg_sh.at[pl.ds(s * NPT, NPT)])

    @pl.when(c == 1)
    def _():
        pltpu.sync_copy(zeros_hbm, deg_sh.at[pl.ds(s * NPT, NPT)])

    pltpu.sync_copy(ones1_hbm, ones_v)
    pltpu.sync_copy(dstr_hbm.at[wid], idx_d)
    plsc.subcore_barrier()

    sems = (sem0, sem1)

    def body(j, carry):
        for p in range(2):
            i = j * 2 + p
            pltpu.async_copy(ones_v, deg_sh.at[idx_d.at[i]], sems[p],
                             add=True)
        for p in range(2):
            i = j * 2 + p
            pltpu.make_async_copy(ones_v, deg_sh.at[idx_d.at[i]],
                                  sems[p]).wait()
        return carry

    lax.fori_loop(0, CPW // 2, body, 0)
    pltpu.sync_copy(ones_v, deg_sh.at[idx_d.at[CPW - 1]], add=True)
    plsc.subcore_barrier()

    pltpu.sync_copy(deg_sh.at[pl.ds(s * NPT, NPT)],
                    deg_out.at[pl.ds(c * NPAD + s * NPT, NPT)])


@functools.partial(
    pl.kernel,
    out_type=jax.ShapeDtypeStruct((2 * NPAD, D), jnp.float32),
    mesh=_mesh,
    scratch_types=[
        pltpu.VMEM((CPWP, KE), jnp.int32),
        pltpu.VMEM((HB, KE), jnp.int32),
        pltpu.VMEM((KE, D), jnp.float32),
        pltpu.VMEM((KE, D), jnp.float32),
        pltpu.VMEM_SHARED((NPAD, D), jnp.float32),
        pltpu.SemaphoreType.DMA,
        pltpu.SemaphoreType.DMA,
        pltpu.SemaphoreType.DMA,
        pltpu.SemaphoreType.DMA,
    ],
)
def _agg(hp_hbm, srcr_hbm, dstr_hbm, zrows_hbm, acc_out,
         idx_s, idx_d, rows0, rows1, acc_sh, sem0, sem1, ssem0, ssem1):
    c = lax.axis_index("c")
    s = lax.axis_index("s")
    wid = s * NC + c

    @pl.when(c == 0)
    def _():
        pltpu.sync_copy(hp_hbm.at[pl.ds(s * NPT, NPT)],
                        acc_sh.at[pl.ds(s * NPT, NPT)])

    @pl.when(c == 1)
    def _():
        pltpu.sync_copy(zrows_hbm, acc_sh.at[pl.ds(s * NPT, NPT)])

    pltpu.sync_copy(srcr_hbm.at[wid], idx_s)
    pltpu.sync_copy(dstr_hbm.at[wid, pl.ds(0, HB)], idx_d)
    plsc.subcore_barrier()

    bufs = (rows0, rows1)
    sems = (sem0, sem1)
    ssems = (ssem0, ssem1)

    def start_g(i, p):
        pltpu.async_copy(hp_hbm.at[idx_s.at[i]], bufs[p], sems[p])

    def wait_g(i, p):
        pltpu.make_async_copy(hp_hbm.at[idx_s.at[i]], bufs[p], sems[p]).wait()

    def start_s(ild, p):
        pltpu.async_copy(bufs[p], acc_sh.at[idx_d.at[ild]], ssems[p],
                         add=True)

    def wait_s(ild, p):
        pltpu.make_async_copy(bufs[p], acc_sh.at[idx_d.at[ild]],
                              ssems[p]).wait()

    start_g(0, 0)
    start_g(1, 1)

    def body0(j, carry):
        for p in range(2):
            i = j * 2 + p
            wait_g(i, p)
            start_s(i, p)
        for p in range(2):
            i = j * 2 + p
            wait_s(i, p)
            start_g(i + 2, p)
        return carry

    lax.fori_loop(0, HB // 2, body0, 0)

    pltpu.sync_copy(dstr_hbm.at[wid, pl.ds(HB, HB)], idx_d)

    def body1(j, carry):
        for p in range(2):
            i = HB + j * 2 + p

            @pl.when(i < CPW)
            def _(i=i, p=p):
                wait_g(i, p)
                start_s(i - HB, p)
        for p in range(2):
            i = HB + j * 2 + p

            @pl.when(i < CPW)
            def _(i=i, p=p):
                wait_s(i - HB, p)

                @pl.when(i + 2 < CPW)
                def _(i=i, p=p):
                    start_g(i + 2, p)
        return carry

    lax.fori_loop(0, (CPW - HB + 1) // 2, body1, 0)
    plsc.subcore_barrier()

    pltpu.sync_copy(acc_sh.at[pl.ds(s * NPT, NPT)],
                    acc_out.at[pl.ds(c * NPAD + s * NPT, NPT)])


def _scale_body(x_ref, w_ref, d0_ref, d1_ref, hp_ref, dinv_ref):
    h = jnp.dot(x_ref[...], w_ref[...], preferred_element_type=jnp.float32)
    dinv = lax.rsqrt(d0_ref[...] + d1_ref[...])
    hp_ref[...] = h * dinv
    dinv_ref[...] = dinv


_RB = 1024


def _scale(x, w, d0, d1):
    return pl.pallas_call(
        _scale_body,
        grid=(NPAD // _RB,),
        in_specs=[
            pl.BlockSpec((_RB, D), lambda i: (i, 0)),
            pl.BlockSpec((D, D), lambda i: (0, 0)),
            pl.BlockSpec((_RB, 1), lambda i: (i, 0)),
            pl.BlockSpec((_RB, 1), lambda i: (i, 0)),
        ],
        out_specs=[
            pl.BlockSpec((_RB, D), lambda i: (i, 0)),
            pl.BlockSpec((_RB, 1), lambda i: (i, 0)),
        ],
        out_shape=[
            jax.ShapeDtypeStruct((NPAD, D), jnp.float32),
            jax.ShapeDtypeStruct((NPAD, 1), jnp.float32),
        ],
    )(x, w, d0, d1)


def _combine_body(a0_ref, a1_ref, dinv_ref, out_ref):
    out_ref[...] = dinv_ref[...] * (a0_ref[...] + a1_ref[...])


def _combine(a0, a1, dinv):
    return pl.pallas_call(
        _combine_body,
        grid=(NPAD // _RB,),
        in_specs=[
            pl.BlockSpec((_RB, D), lambda i: (i, 0)),
            pl.BlockSpec((_RB, D), lambda i: (i, 0)),
            pl.BlockSpec((_RB, 1), lambda i: (i, 0)),
        ],
        out_specs=pl.BlockSpec((_RB, D), lambda i: (i, 0)),
        out_shape=jax.ShapeDtypeStruct((NPAD, D), jnp.float32),
    )(a0, a1, dinv)


def kernel(node_feature, edge_index, W):
    srcr = jnp.pad(edge_index[0].reshape(NW, CPW, KE),
                   ((0, 0), (0, CPWP - CPW), (0, 0)))
    dstr = jnp.pad(edge_index[1].reshape(NW, CPW, KE),
                   ((0, 0), (0, CPWP - CPW), (0, 0)))

    ones_s = jnp.ones((NPT,), jnp.float32)
    zeros_s = jnp.zeros((NPT,), jnp.float32)
    ones1_s = jnp.ones((KE,), jnp.float32)
    zrows_s = jnp.zeros((NPT, D), jnp.float32)

    deg2 = _deg(dstr, ones_s, zeros_s, ones1_s).reshape(2, NPAD, 1)

    x_pad = jnp.pad(node_feature, ((0, NPAD - N), (0, 0)))
    hp, dinv = _scale(x_pad, W, deg2[0], deg2[1])

    acc2 = _agg(hp, srcr, dstr, zrows_s).reshape(2, NPAD, D)
    out = _combine(acc2[0], acc2[1], dinv)
    return out[:N]


# speedup vs baseline: 1.2773x; 1.2773x over previous
import functools

import jax
import jax.numpy as jnp
from jax import lax
from jax.experimental import pallas as pl
from jax.experimental.pallas import tpu as pltpu
from jax.experimental.pallas import tpu_sc as plsc

N = 10000
E = 320000
D = 128

NC = 2
NS = 16
NW = NC * NS
NPAD = 10240
NPT = NPAD // NS
KE = 80
EPW = E // NW
CPW = EPW // KE
CPWP = 128
HB = 64

_mesh = plsc.VectorSubcoreMesh(core_axis_name="c", subcore_axis_name="s")


@functools.partial(
    pl.kernel,
    out_type=jax.ShapeDtypeStruct((2 * NPAD,), jnp.float32),
    mesh=_mesh,
    scratch_types=[
        pltpu.VMEM((CPWP, KE), jnp.int32),
        pltpu.VMEM((KE,), jnp.float32),
        pltpu.VMEM_SHARED((NPAD,), jnp.float32),
        pltpu.SemaphoreType.DMA,
        pltpu.SemaphoreType.DMA,
    ],
)
def _deg(edge_hbm, ones_hbm, zeros_hbm, ones1_hbm, deg_out,
         idx_d, ones_v, deg_sh, sem0, sem1):
    c = lax.axis_index("c")
    s = lax.axis_index("s")
    wid = s * NC + c

    @pl.when(c == 0)
    def _():
        pltpu.sync_copy(ones_hbm, deg_sh.at[pl.ds(s * NPT, NPT)])

    @pl.when(c == 1)
    def _():
        pltpu.sync_copy(zeros_hbm, deg_sh.at[pl.ds(s * NPT, NPT)])

    pltpu.sync_copy(ones1_hbm, ones_v)
    pltpu.sync_copy(edge_hbm.at[1, wid], idx_d)
    plsc.subcore_barrier()

    sems = (sem0, sem1)

    def body(j, carry):
        for p in range(2):
            i = j * 2 + p
            pltpu.async_copy(ones_v, deg_sh.at[idx_d.at[i]], sems[p],
                             add=True)
        for p in range(2):
            i = j * 2 + p
            pltpu.make_async_copy(ones_v, deg_sh.at[idx_d.at[i]],
                                  sems[p]).wait()
        return carry

    lax.fori_loop(0, CPW // 2, body, 0)
    pltpu.sync_copy(ones_v, deg_sh.at[idx_d.at[CPW - 1]], add=True)
    plsc.subcore_barrier()

    pltpu.sync_copy(deg_sh.at[pl.ds(s * NPT, NPT)],
                    deg_out.at[pl.ds(c * NPAD + s * NPT, NPT)])


@functools.partial(
    pl.kernel,
    out_type=jax.ShapeDtypeStruct((2 * NPAD, D), jnp.float32),
    mesh=_mesh,
    scratch_types=[
        pltpu.VMEM((CPWP, KE), jnp.int32),
        pltpu.VMEM((HB, KE), jnp.int32),
        pltpu.VMEM((KE, D), jnp.float32),
        pltpu.VMEM((KE, D), jnp.float32),
        pltpu.VMEM_SHARED((NPAD, D), jnp.float32),
        pltpu.SemaphoreType.DMA,
        pltpu.SemaphoreType.DMA,
    ],
)
def _agg(hp_hbm, edge_hbm, zrows_hbm, acc_out,
         idx_s, idx_d, rows0, rows1, acc_sh, sem0, sem1):
    c = lax.axis_index("c")
    s = lax.axis_index("s")
    wid = s * NC + c

    @pl.when(c == 0)
    def _():
        pltpu.sync_copy(hp_hbm.at[pl.ds(s * NPT, NPT)],
                        acc_sh.at[pl.ds(s * NPT, NPT)])

    @pl.when(c == 1)
    def _():
        pltpu.sync_copy(zrows_hbm, acc_sh.at[pl.ds(s * NPT, NPT)])

    pltpu.sync_copy(edge_hbm.at[0, wid], idx_s)
    pltpu.sync_copy(edge_hbm.at[1, wid, pl.ds(0, HB)], idx_d)
    plsc.subcore_barrier()

    bufs = (rows0, rows1)
    sems = (sem0, sem1)

    def start_g(i, p):
        pltpu.async_copy(hp_hbm.at[idx_s.at[i]], bufs[p], sems[p])

    def wait_g(i, p):
        pltpu.make_async_copy(hp_hbm.at[idx_s.at[i]], bufs[p], sems[p]).wait()

    start_g(0, 0)
    start_g(1, 1)

    def body0(j, carry):
        for p in range(2):
            i = j * 2 + p
            wait_g(i, p)
            pltpu.sync_copy(bufs[p], acc_sh.at[idx_d.at[i]], add=True)
            start_g(i + 2, p)
        return carry

    lax.fori_loop(0, HB // 2, body0, 0)

    pltpu.sync_copy(edge_hbm.at[1, wid, pl.ds(HB, HB)], idx_d)

    def body1(j, carry):
        for p in range(2):
            i = HB + j * 2 + p

            @pl.when(i < CPW)
            def _(i=i, p=p):
                wait_g(i, p)
                pltpu.sync_copy(bufs[p], acc_sh.at[idx_d.at[i - HB]],
                                add=True)

                @pl.when(i + 2 < CPW)
                def _(i=i, p=p):
                    start_g(i + 2, p)
        return carry

    lax.fori_loop(0, (CPW - HB + 1) // 2, body1, 0)
    plsc.subcore_barrier()

    pltpu.sync_copy(acc_sh.at[pl.ds(s * NPT, NPT)],
                    acc_out.at[pl.ds(c * NPAD + s * NPT, NPT)])


def _scale_body(x_ref, w_ref, d0_ref, d1_ref, hp_ref, dinv_ref):
    h = jnp.dot(x_ref[...], w_ref[...], preferred_element_type=jnp.float32)
    dinv = lax.rsqrt(d0_ref[...] + d1_ref[...])
    hp_ref[...] = h * dinv
    dinv_ref[...] = dinv


_NBLK = NPAD // 1024


_RB = 1024


def _scale(x, w, deg_col):
    return pl.pallas_call(
        _scale_body,
        grid=(NPAD // _RB,),
        in_specs=[
            pl.BlockSpec((_RB, D), lambda i: (i, 0)),
            pl.BlockSpec((D, D), lambda i: (0, 0)),
            pl.BlockSpec((_RB, 1), lambda i: (i, 0)),
            pl.BlockSpec((_RB, 1), lambda i: (_NBLK + i, 0)),
        ],
        out_specs=[
            pl.BlockSpec((_RB, D), lambda i: (i, 0)),
            pl.BlockSpec((_RB, 1), lambda i: (i, 0)),
        ],
        out_shape=[
            jax.ShapeDtypeStruct((NPAD, D), jnp.float32),
            jax.ShapeDtypeStruct((NPAD, 1), jnp.float32),
        ],
    )(x, w, deg_col, deg_col)


def _combine_body(a0_ref, a1_ref, dinv_ref, out_ref):
    out_ref[...] = dinv_ref[...] * (a0_ref[...] + a1_ref[...])


def _combine(acc_flat, dinv):
    return pl.pallas_call(
        _combine_body,
        grid=(NPAD // _RB,),
        in_specs=[
            pl.BlockSpec((_RB, D), lambda i: (i, 0)),
            pl.BlockSpec((_RB, D), lambda i: (_NBLK + i, 0)),
            pl.BlockSpec((_RB, 1), lambda i: (i, 0)),
        ],
        out_specs=pl.BlockSpec((_RB, D), lambda i: (i, 0)),
        out_shape=jax.ShapeDtypeStruct((NPAD, D), jnp.float32),
    )(acc_flat, acc_flat, dinv)


def kernel(node_feature, edge_index, W):
    edge4 = jnp.pad(edge_index.reshape(2, NW, CPW, KE),
                    ((0, 0), (0, 0), (0, CPWP - CPW), (0, 0)))

    ones_s = jnp.ones((NPT,), jnp.float32)
    zeros_s = jnp.zeros((NPT,), jnp.float32)
    ones1_s = jnp.ones((KE,), jnp.float32)
    zrows_s = jnp.zeros((NPT, D), jnp.float32)

    deg_col = _deg(edge4, ones_s, zeros_s, ones1_s).reshape(2 * NPAD, 1)

    x_pad = jnp.pad(node_feature, ((0, NPAD - N), (0, 0)))
    hp, dinv = _scale(x_pad, W, deg_col)

    acc_flat = _agg(hp, edge4, zrows_s)
    out = _combine(acc_flat, dinv)
    return out[:N]


# speedup vs baseline: 1.3908x; 1.0889x over previous
import functools

import jax
import jax.numpy as jnp
from jax import lax
from jax.experimental import pallas as pl
from jax.experimental.pallas import tpu as pltpu
from jax.experimental.pallas import tpu_sc as plsc

N = 10000
E = 320000
D = 128

NC = 2
NS = 16
NW = NC * NS
NPAD = 10240
NPT = NPAD // NS
KE = 80
EPW = E // NW
CPW = EPW // KE
CPWP = 128
HB = 64

_mesh = plsc.VectorSubcoreMesh(core_axis_name="c", subcore_axis_name="s")


@functools.partial(
    pl.kernel,
    out_type=jax.ShapeDtypeStruct((2 * NPAD,), jnp.float32),
    mesh=_mesh,
    scratch_types=[
        pltpu.VMEM((CPWP, KE), jnp.int32),
        pltpu.VMEM((KE,), jnp.float32),
        pltpu.VMEM_SHARED((NPAD,), jnp.float32),
        pltpu.SemaphoreType.DMA,
        pltpu.SemaphoreType.DMA,
    ],
)
def _deg(edge_hbm, ones_hbm, zeros_hbm, ones1_hbm, deg_out,
         idx_d, ones_v, deg_sh, sem0, sem1):
    c = lax.axis_index("c")
    s = lax.axis_index("s")
    wid = s * NC + c

    @pl.when(c == 0)
    def _():
        pltpu.sync_copy(ones_hbm, deg_sh.at[pl.ds(s * NPT, NPT)])

    @pl.when(c == 1)
    def _():
        pltpu.sync_copy(zeros_hbm, deg_sh.at[pl.ds(s * NPT, NPT)])

    pltpu.sync_copy(ones1_hbm, ones_v)
    pltpu.sync_copy(edge_hbm.at[1, wid], idx_d)
    plsc.subcore_barrier()

    sems = (sem0, sem1)

    def body(j, carry):
        for p in range(2):
            i = j * 2 + p
            pltpu.async_copy(ones_v, deg_sh.at[idx_d.at[i]], sems[p],
                             add=True)
        for p in range(2):
            i = j * 2 + p
            pltpu.make_async_copy(ones_v, deg_sh.at[idx_d.at[i]],
                                  sems[p]).wait()
        return carry

    lax.fori_loop(0, CPW // 2, body, 0)
    pltpu.sync_copy(ones_v, deg_sh.at[idx_d.at[CPW - 1]], add=True)
    plsc.subcore_barrier()

    pltpu.sync_copy(deg_sh.at[pl.ds(s * NPT, NPT)],
                    deg_out.at[pl.ds(c * NPAD + s * NPT, NPT)])


@functools.partial(
    pl.kernel,
    out_type=jax.ShapeDtypeStruct((2 * NPAD, D), jnp.float32),
    mesh=_mesh,
    scratch_types=[
        pltpu.VMEM((CPWP, KE), jnp.int32),
        pltpu.VMEM((HB, KE), jnp.int32),
        pltpu.VMEM((KE, D), jnp.float32),
        pltpu.VMEM((KE, D), jnp.float32),
        pltpu.VMEM_SHARED((NPAD, D), jnp.float32),
        pltpu.SemaphoreType.DMA,
        pltpu.SemaphoreType.DMA,
    ],
)
def _agg(hp_hbm, edge_hbm, zrows_hbm, acc_out,
         idx_s, idx_d, rows0, rows1, acc_sh, sem0, sem1):
    c = lax.axis_index("c")
    s = lax.axis_index("s")
    wid = s * NC + c

    @pl.when(c == 0)
    def _():
        pltpu.sync_copy(hp_hbm.at[pl.ds(s * NPT, NPT)],
                        acc_sh.at[pl.ds(s * NPT, NPT)])

    @pl.when(c == 1)
    def _():
        pltpu.sync_copy(zrows_hbm, acc_sh.at[pl.ds(s * NPT, NPT)])

    pltpu.sync_copy(edge_hbm.at[0, wid], idx_s)
    pltpu.sync_copy(edge_hbm.at[1, wid, pl.ds(0, HB)], idx_d)
    plsc.subcore_barrier()

    bufs = (rows0, rows1)
    sems = (sem0, sem1)

    def start_g(i, p):
        pltpu.async_copy(hp_hbm.at[idx_s.at[i]], bufs[p], sems[p])

    def wait_g(i, p):
        pltpu.make_async_copy(hp_hbm.at[idx_s.at[i]], bufs[p], sems[p]).wait()

    start_g(0, 0)
    start_g(1, 1)

    def body0(j, carry):
        for p in range(2):
            i = j * 2 + p
            wait_g(i, p)
            start_g(i + 2, p)
        return carry

    lax.fori_loop(0, HB // 2, body0, 0)

    pltpu.sync_copy(edge_hbm.at[1, wid, pl.ds(HB, HB)], idx_d)

    def body1(j, carry):
        for p in range(2):
            i = HB + j * 2 + p

            @pl.when(i < CPW)
            def _(i=i, p=p):
                wait_g(i, p)

                @pl.when(i + 2 < CPW)
                def _(i=i, p=p):
                    start_g(i + 2, p)
        return carry

    lax.fori_loop(0, (CPW - HB + 1) // 2, body1, 0)
    plsc.subcore_barrier()

    pltpu.sync_copy(acc_sh.at[pl.ds(s * NPT, NPT)],
                    acc_out.at[pl.ds(c * NPAD + s * NPT, NPT)])


def _scale_body(x_ref, w_ref, d0_ref, d1_ref, hp_ref, dinv_ref):
    h = jnp.dot(x_ref[...], w_ref[...], preferred_element_type=jnp.float32)
    dinv = lax.rsqrt(d0_ref[...] + d1_ref[...])
    hp_ref[...] = h * dinv
    dinv_ref[...] = dinv


_NBLK = NPAD // 1024


_RB = 1024


def _scale(x, w, deg_col):
    return pl.pallas_call(
        _scale_body,
        grid=(NPAD // _RB,),
        in_specs=[
            pl.BlockSpec((_RB, D), lambda i: (i, 0)),
            pl.BlockSpec((D, D), lambda i: (0, 0)),
            pl.BlockSpec((_RB, 1), lambda i: (i, 0)),
            pl.BlockSpec((_RB, 1), lambda i: (_NBLK + i, 0)),
        ],
        out_specs=[
            pl.BlockSpec((_RB, D), lambda i: (i, 0)),
            pl.BlockSpec((_RB, 1), lambda i: (i, 0)),
        ],
        out_shape=[
            jax.ShapeDtypeStruct((NPAD, D), jnp.float32),
            jax.ShapeDtypeStruct((NPAD, 1), jnp.float32),
        ],
    )(x, w, deg_col, deg_col)


def _combine_body(a0_ref, a1_ref, dinv_ref, out_ref):
    out_ref[...] = dinv_ref[...] * (a0_ref[...] + a1_ref[...])


def _combine(acc_flat, dinv):
    return pl.pallas_call(
        _combine_body,
        grid=(NPAD // _RB,),
        in_specs=[
            pl.BlockSpec((_RB, D), lambda i: (i, 0)),
            pl.BlockSpec((_RB, D), lambda i: (_NBLK + i, 0)),
            pl.BlockSpec((_RB, 1), lambda i: (i, 0)),
        ],
        out_specs=pl.BlockSpec((_RB, D), lambda i: (i, 0)),
        out_shape=jax.ShapeDtypeStruct((NPAD, D), jnp.float32),
    )(acc_flat, acc_flat, dinv)


def kernel(node_feature, edge_index, W):
    edge4 = jnp.pad(edge_index.reshape(2, NW, CPW, KE),
                    ((0, 0), (0, 0), (0, CPWP - CPW), (0, 0)))

    ones_s = jnp.ones((NPT,), jnp.float32)
    zeros_s = jnp.zeros((NPT,), jnp.float32)
    ones1_s = jnp.ones((KE,), jnp.float32)
    zrows_s = jnp.zeros((NPT, D), jnp.float32)

    deg_col = _deg(edge4, ones_s, zeros_s, ones1_s).reshape(2 * NPAD, 1)

    x_pad = jnp.pad(node_feature, ((0, NPAD - N), (0, 0)))
    hp, dinv = _scale(x_pad, W, deg_col)

    acc_flat = _agg(hp, edge4, zrows_s)
    out = _combine(acc_flat, dinv)
    return out[:N]


# speedup vs baseline: 1.4283x; 1.0269x over previous
import functools

import jax
import jax.numpy as jnp
from jax import lax
from jax.experimental import pallas as pl
from jax.experimental.pallas import tpu as pltpu
from jax.experimental.pallas import tpu_sc as plsc

N = 10000
E = 320000
D = 128

NC = 2
NS = 16
NW = NC * NS
NPAD = 10240
NPT = NPAD // NS
KE = 80
EPW = E // NW
CPW = EPW // KE
CPWP = 136
HB = 64
ISB = 72

_mesh = plsc.VectorSubcoreMesh(core_axis_name="c", subcore_axis_name="s")


@functools.partial(
    pl.kernel,
    out_type=jax.ShapeDtypeStruct((2 * NPAD,), jnp.float32),
    mesh=_mesh,
    scratch_types=[
        pltpu.VMEM((CPWP, KE), jnp.int32),
        pltpu.VMEM((KE,), jnp.float32),
        pltpu.VMEM_SHARED((NPAD,), jnp.float32),
        pltpu.SemaphoreType.DMA,
        pltpu.SemaphoreType.DMA,
    ],
)
def _deg(edge_hbm, ones_hbm, zeros_hbm, ones1_hbm, deg_out,
         idx_d, ones_v, deg_sh, sem0, sem1):
    c = lax.axis_index("c")
    s = lax.axis_index("s")
    wid = s * NC + c

    @pl.when(c == 0)
    def _():
        pltpu.sync_copy(ones_hbm, deg_sh.at[pl.ds(s * NPT, NPT)])

    @pl.when(c == 1)
    def _():
        pltpu.sync_copy(zeros_hbm, deg_sh.at[pl.ds(s * NPT, NPT)])

    pltpu.sync_copy(ones1_hbm, ones_v)
    pltpu.sync_copy(edge_hbm.at[1, wid], idx_d)
    plsc.subcore_barrier()

    sems = (sem0, sem1)

    def body(j, carry):
        for p in range(2):
            i = j * 2 + p
            pltpu.async_copy(ones_v, deg_sh.at[idx_d.at[i]], sems[p],
                             add=True)
        for p in range(2):
            i = j * 2 + p
            pltpu.make_async_copy(ones_v, deg_sh.at[idx_d.at[i]],
                                  sems[p]).wait()
        return carry

    lax.fori_loop(0, CPW // 2, body, 0)
    pltpu.sync_copy(ones_v, deg_sh.at[idx_d.at[CPW - 1]], add=True)
    plsc.subcore_barrier()

    pltpu.sync_copy(deg_sh.at[pl.ds(s * NPT, NPT)],
                    deg_out.at[pl.ds(c * NPAD + s * NPT, NPT)])


@functools.partial(
    pl.kernel,
    out_type=jax.ShapeDtypeStruct((2 * NPAD, D), jnp.float32),
    mesh=_mesh,
    scratch_types=[
        pltpu.VMEM((ISB, KE), jnp.int32),
        pltpu.VMEM((HB, KE), jnp.int32),
        pltpu.VMEM((KE, D), jnp.float32),
        pltpu.VMEM((KE, D), jnp.float32),
        pltpu.VMEM((KE, D), jnp.float32),
        pltpu.VMEM_SHARED((NPAD, D), jnp.float32),
        pltpu.SemaphoreType.DMA,
        pltpu.SemaphoreType.DMA,
        pltpu.SemaphoreType.DMA,
    ],
)
def _agg(hp_hbm, edge_hbm, zrows_hbm, acc_out,
         idx_s, idx_d, rows0, rows1, rows2, acc_sh, sem0, sem1, sem2):
    c = lax.axis_index("c")
    s = lax.axis_index("s")
    wid = s * NC + c

    @pl.when(c == 0)
    def _():
        pltpu.sync_copy(hp_hbm.at[pl.ds(s * NPT, NPT)],
                        acc_sh.at[pl.ds(s * NPT, NPT)])

    @pl.when(c == 1)
    def _():
        pltpu.sync_copy(zrows_hbm, acc_sh.at[pl.ds(s * NPT, NPT)])

    pltpu.sync_copy(edge_hbm.at[0, wid, pl.ds(0, ISB)], idx_s)
    pltpu.sync_copy(edge_hbm.at[1, wid, pl.ds(0, HB)], idx_d)
    plsc.subcore_barrier()

    bufs = (rows0, rows1, rows2)
    sems = (sem0, sem1, sem2)

    def start_g(l, p):
        pltpu.async_copy(hp_hbm.at[idx_s.at[l]], bufs[p], sems[p])

    def wait_g(l, p):
        pltpu.make_async_copy(hp_hbm.at[idx_s.at[l]], bufs[p], sems[p]).wait()

    for p in range(3):
        start_g(p, p)

    def body0(j, carry):
        for p in range(3):
            i = j * 3 + p

            @pl.when(i < HB)
            def _(i=i, p=p):
                wait_g(i, p)
                pltpu.sync_copy(bufs[p], acc_sh.at[idx_d.at[i]], add=True)
                start_g(i + 3, p)
        return carry

    lax.fori_loop(0, (HB + 2 + 2) // 3, body0, 0)

    for k in (HB, HB + 1, HB + 2):
        wait_g(k, k % 3)
    pltpu.sync_copy(edge_hbm.at[1, wid, pl.ds(HB, HB)], idx_d)
    pltpu.sync_copy(edge_hbm.at[0, wid, pl.ds(HB, ISB)], idx_s)
    for k in (HB, HB + 1, HB + 2):
        pltpu.sync_copy(bufs[k % 3], acc_sh.at[idx_d.at[k - HB]], add=True)
        start_g(k + 3 - HB, k % 3)

    def body1(j, carry):
        for p in range(3):
            i = HB + 3 + j * 3 + p
            b = (HB + 3 + p) % 3

            @pl.when(i < CPW)
            def _(i=i, b=b):
                wait_g(i - HB, b)
                pltpu.sync_copy(bufs[b], acc_sh.at[idx_d.at[i - HB]],
                                add=True)

                @pl.when(i + 3 < CPW)
                def _(i=i, b=b):
                    start_g(i + 3 - HB, b)
        return carry

    lax.fori_loop(0, (CPW - HB - 3 + 2) // 3, body1, 0)
    plsc.subcore_barrier()

    pltpu.sync_copy(acc_sh.at[pl.ds(s * NPT, NPT)],
                    acc_out.at[pl.ds(c * NPAD + s * NPT, NPT)])


def _scale_body(x_ref, w_ref, d0_ref, d1_ref, hp_ref, dinv_ref):
    h = jnp.dot(x_ref[...], w_ref[...], preferred_element_type=jnp.float32)
    dinv = lax.rsqrt(d0_ref[...] + d1_ref[...])
    hp_ref[...] = h * dinv
    dinv_ref[...] = dinv


_NBLK = NPAD // 1024


_RB = 1024


def _scale(x, w, deg_col):
    return pl.pallas_call(
        _scale_body,
        grid=(NPAD // _RB,),
        in_specs=[
            pl.BlockSpec((_RB, D), lambda i: (i, 0)),
            pl.BlockSpec((D, D), lambda i: (0, 0)),
            pl.BlockSpec((_RB, 1), lambda i: (i, 0)),
            pl.BlockSpec((_RB, 1), lambda i: (_NBLK + i, 0)),
        ],
        out_specs=[
            pl.BlockSpec((_RB, D), lambda i: (i, 0)),
            pl.BlockSpec((_RB, 1), lambda i: (i, 0)),
        ],
        out_shape=[
            jax.ShapeDtypeStruct((NPAD, D), jnp.float32),
            jax.ShapeDtypeStruct((NPAD, 1), jnp.float32),
        ],
    )(x, w, deg_col, deg_col)


def _combine_body(a0_ref, a1_ref, dinv_ref, out_ref):
    out_ref[...] = dinv_ref[...] * (a0_ref[...] + a1_ref[...])


def _combine(acc_flat, dinv):
    return pl.pallas_call(
        _combine_body,
        grid=(NPAD // _RB,),
        in_specs=[
            pl.BlockSpec((_RB, D), lambda i: (i, 0)),
            pl.BlockSpec((_RB, D), lambda i: (_NBLK + i, 0)),
            pl.BlockSpec((_RB, 1), lambda i: (i, 0)),
        ],
        out_specs=pl.BlockSpec((_RB, D), lambda i: (i, 0)),
        out_shape=jax.ShapeDtypeStruct((NPAD, D), jnp.float32),
    )(acc_flat, acc_flat, dinv)


def kernel(node_feature, edge_index, W):
    edge4 = jnp.pad(edge_index.reshape(2, NW, CPW, KE),
                    ((0, 0), (0, 0), (0, CPWP - CPW), (0, 0)))

    ones_s = jnp.ones((NPT,), jnp.float32)
    zeros_s = jnp.zeros((NPT,), jnp.float32)
    ones1_s = jnp.ones((KE,), jnp.float32)
    zrows_s = jnp.zeros((NPT, D), jnp.float32)

    deg_col = _deg(edge4, ones_s, zeros_s, ones1_s).reshape(2 * NPAD, 1)

    x_pad = jnp.pad(node_feature, ((0, NPAD - N), (0, 0)))
    hp, dinv = _scale(x_pad, W, deg_col)

    acc_flat = _agg(hp, edge4, zrows_s)
    out = _combine(acc_flat, dinv)
    return out[:N]
